# baseline (speedup 1.0000x reference)
import functools

import jax
import jax.numpy as jnp
from jax import lax
from jax.experimental import pallas as pl
from jax.experimental.pallas import tpu as pltpu
from jax.experimental.pallas import tpu_sc as plsc

B = 128
N = 32768
M = N // 2

_info = plsc.get_sparse_core_info()
_NC, _NS, _L = _info.num_cores, _info.num_subcores, _info.num_lanes
_NW = _NC * _NS
_ROWS_PER_W = B // _NW

CHUNK = N // 2
_CPR = N // CHUNK
_NCH = _ROWS_PER_W * _CPR
RING = 4


def _sc_body(in_hbm, out_hbm, *scratch):
    in_bufs = scratch[0:RING]
    out_bufs = scratch[RING:2 * RING]
    in_sems = scratch[2 * RING:3 * RING]
    out_sems = scratch[3 * RING:4 * RING]

    wid = lax.axis_index("s") * _NC + lax.axis_index("c")
    base_row = wid * _ROWS_PER_W
    lane = lax.iota(jnp.int32, _L)

    def chunk_src(c):
        row = base_row + c // _CPR
        return in_hbm.at[row, pl.ds((c % _CPR) * CHUNK, CHUNK)]

    def chunk_dst(c):
        row = base_row + c // _CPR
        return out_hbm.at[row, pl.ds((c % _CPR) * (CHUNK // 2), CHUNK // 2)]

    def gather(src, dst):
        @plsc.parallel_loop(0, CHUNK // 2 // _L, unroll=2)
        def _(j):
            idx = (2 * _L) * j + 2 * lane
            dst[pl.ds(j * _L, _L)] = plsc.load_gather(src, [idx])

    in_cp = {}
    out_cp = {}
    for c in range(RING - 1):
        in_cp[c] = pltpu.async_copy(chunk_src(c), in_bufs[c % RING],
                                    in_sems[c % RING])
    for c in range(_NCH):
        p = c % RING
        in_cp[c].wait()
        nxt = c + RING - 1
        if nxt < _NCH:
            in_cp[nxt] = pltpu.async_copy(chunk_src(nxt), in_bufs[nxt % RING],
                                          in_sems[nxt % RING])
        if c >= RING:
            out_cp[c - RING].wait()
        gather(in_bufs[p], out_bufs[p])
        out_cp[c] = pltpu.async_copy(out_bufs[p], chunk_dst(c), out_sems[p])
    for c in range(_NCH - RING, _NCH):
        out_cp[c].wait()


@jax.jit
def kernel(inputs):
    mesh = plsc.VectorSubcoreMesh(core_axis_name="c", subcore_axis_name="s")
    f = functools.partial(
        pl.kernel,
        mesh=mesh,
        out_type=jax.ShapeDtypeStruct((B, M), jnp.float32),
        scratch_types=(
            [pltpu.VMEM((CHUNK,), jnp.float32) for _ in range(RING)]
            + [pltpu.VMEM((CHUNK // 2,), jnp.float32) for _ in range(RING)]
            + [pltpu.SemaphoreType.DMA for _ in range(2 * RING)]
        ),
        compiler_params=pltpu.CompilerParams(needs_layout_passes=False),
    )(_sc_body)
    return f(inputs)


# speedup vs baseline: 1.1273x; 1.1273x over previous
import functools

import jax
import jax.numpy as jnp
from jax import lax
from jax.experimental import pallas as pl
from jax.experimental.pallas import tpu as pltpu
from jax.experimental.pallas import tpu_sc as plsc

B = 128
N = 32768
M = N // 2

_info = plsc.get_sparse_core_info()
_NC, _NS, _L = _info.num_cores, _info.num_subcores, _info.num_lanes
_NW = _NC * _NS
_ROWS_PER_W = B // _NW

CHUNK = N // 2
_CPR = N // CHUNK
_NCH = _ROWS_PER_W * _CPR
RING = 4


def _sc_body(in_hbm, out_hbm, *scratch):
    in_bufs = scratch[0:RING]
    out_bufs = scratch[RING:2 * RING]
    in_sems = scratch[2 * RING:3 * RING]
    out_sems = scratch[3 * RING:4 * RING]

    wid = lax.axis_index("s") * _NC + lax.axis_index("c")
    base_row = wid * _ROWS_PER_W
    lane = lax.iota(jnp.int32, _L)

    def chunk_src(c):
        row = base_row + c // _CPR
        return in_hbm.at[row, pl.ds((c % _CPR) * CHUNK, CHUNK)]

    def chunk_dst(c):
        row = base_row + c // _CPR
        return out_hbm.at[row, pl.ds((c % _CPR) * (CHUNK // 2), CHUNK // 2)]

    def gather(src, dst):
        @plsc.parallel_loop(0, CHUNK // 2 // _L, unroll=8)
        def _(j):
            idx = (2 * _L) * j + 2 * lane
            dst[pl.ds(j * _L, _L)] = plsc.load_gather(src, [idx])

    in_cp = {}
    out_cp = {}
    for c in range(RING - 1):
        in_cp[c] = pltpu.async_copy(chunk_src(c), in_bufs[c % RING],
                                    in_sems[c % RING])
    for c in range(_NCH):
        p = c % RING
        in_cp[c].wait()
        nxt = c + RING - 1
        if nxt < _NCH:
            in_cp[nxt] = pltpu.async_copy(chunk_src(nxt), in_bufs[nxt % RING],
                                          in_sems[nxt % RING])
        if c >= RING:
            out_cp[c - RING].wait()
        gather(in_bufs[p], out_bufs[p])
        out_cp[c] = pltpu.async_copy(out_bufs[p], chunk_dst(c), out_sems[p])
    for c in range(_NCH - RING, _NCH):
        out_cp[c].wait()


@jax.jit
def kernel(inputs):
    mesh = plsc.VectorSubcoreMesh(core_axis_name="c", subcore_axis_name="s")
    f = functools.partial(
        pl.kernel,
        mesh=mesh,
        out_type=jax.ShapeDtypeStruct((B, M), jnp.float32),
        scratch_types=(
            [pltpu.VMEM((CHUNK,), jnp.float32) for _ in range(RING)]
            + [pltpu.VMEM((CHUNK // 2,), jnp.float32) for _ in range(RING)]
            + [pltpu.SemaphoreType.DMA for _ in range(2 * RING)]
        ),
        compiler_params=pltpu.CompilerParams(needs_layout_passes=False),
    )(_sc_body)
    return f(inputs)
